# baseline (speedup 1.0000x reference)
import numpy as np
import jax
import jax.numpy as jnp
from jax import lax
from jax.experimental import pallas as pl
from jax.experimental.pallas import tpu as pltpu
from jax.experimental.pallas import tpu_sc as plsc

N = 100000
E = 6400000
G = 1000
C = 8
L = 8
H = 128
T = 6
EPS = 1e-5

NC = 2
NS = 16
NW = NC * NS

CHUNK = 80
RT = N // NS
ET = E // NW
TILE_ROWS = ET // CHUNK
BULK = 50
NBULK = TILE_ROWS // BULK
NBUF = 5

SUB = 250
NSUB = RT // SUB
VPS = SUB // 2

PCHUNK = 64
NP = ((N + NW * PCHUNK - 1) // (NW * PCHUNK)) * (NW * PCHUNK)
PT = NP // NW
PB = PT // PCHUNK
GP = 1024

_MESH = plsc.VectorSubcoreMesh(core_axis_name="c", subcore_axis_name="s",
                               num_cores=NC, num_subcores=NS)
_SC_PARAMS = pltpu.CompilerParams(use_tc_tiling_on_sc=False,
                                  needs_layout_passes=False)

F32 = jnp.float32
I32 = jnp.int32


def _splat(v):
    return jnp.full((16,), v, I32)


def _rsqrt_scalar(x):
    xi = lax.bitcast_convert_type(x, I32)
    y = lax.bitcast_convert_type(jnp.int32(0x5F3759DF) - (xi >> 1), F32)
    for _ in range(4):
        y = y * (1.5 - 0.5 * x * y * y)
    return y


def _rsqrt(x):
    xi = plsc.bitcast(x, I32)
    y = plsc.bitcast(jnp.int32(0x5F3759DF) - (xi >> 1), F32)
    for _ in range(4):
        y = y * (1.5 - 0.5 * x * y * y)
    return y


def _edge_phase(c, s, src_hbm, dst_hbm, sxs, sacc, sbuf, dbuf, gbuf,
                gsem, ssem, out_hbm):
    base = (c * NS + s) * TILE_ROWS

    @pl.loop(0, NBULK)
    def _bulk(g):
        rb = base + g * BULK
        pltpu.sync_copy(src_hbm.at[pl.ds(rb, BULK)], sbuf)
        pltpu.sync_copy(dst_hbm.at[pl.ds(rb, BULK)], dbuf)

        @pl.loop(0, BULK, step=NBUF)
        def _grp(j0):
            gd = [
                pltpu.async_copy(sxs.at[sbuf.at[j0 + b]], gbuf.at[b],
                                 gsem.at[b])
                for b in range(NBUF)
            ]
            sd = []
            for b in range(NBUF):
                gd[b].wait()
                sd.append(
                    pltpu.async_copy(gbuf.at[b], sacc.at[dbuf.at[j0 + b]],
                                     ssem.at[b], add=True))
            for d in sd:
                d.wait()

    plsc.subcore_barrier()
    row0 = s * RT
    pltpu.sync_copy(sacc.at[pl.ds(row0, RT)],
                    out_hbm.at[pl.ds(c * N + row0, RT)])


_EDGE_SCRATCH = [
    pltpu.VMEM_SHARED((N, C), F32),
    pltpu.VMEM_SHARED((N, C), F32),
    pltpu.VMEM_SHARED((2 * NS, 16), F32),
    pltpu.VMEM((SUB, C), F32),
    pltpu.VMEM((SUB, C), F32),
    pltpu.VMEM((SUB, C), F32),
    pltpu.VMEM((SUB, C), F32),
    pltpu.VMEM((SUB, C), F32),
    pltpu.VMEM((11, C), F32),
    pltpu.VMEM((2, 16), F32),
    pltpu.VMEM((2 * NS, 16), F32),
    pltpu.VMEM((BULK, CHUNK), I32),
    pltpu.VMEM((BULK, CHUNK), I32),
    pltpu.VMEM((NBUF, CHUNK, C), F32),
    pltpu.SemaphoreType.DMA((NBUF,)),
    pltpu.SemaphoreType.DMA((NBUF,)),
]


def _stats_and_weights(s, acc1, acc2, colp, pbuf, stb, astb, sstat):
    stb[0] = acc1
    stb[1] = acc2
    pltpu.sync_copy(stb, sstat.at[pl.ds(2 * s, 2)])
    plsc.subcore_barrier()
    pltpu.sync_copy(sstat, astb)
    g1 = astb[0]
    g2 = astb[1]
    for t in range(1, NS):
        g1 = g1 + astb[2 * t]
        g2 = g2 + astb[2 * t + 1]
    gampat = plsc.load_gather(pbuf, [_splat(0), colp])
    betpat = plsc.load_gather(pbuf, [_splat(1), colp])
    cpat = jnp.zeros((16,), F32)
    wpats = []
    for ch in range(C):
        su = g1[ch] + g1[ch + 8]
        sq = g2[ch] + g2[ch + 8]
        mu = su * (1.0 / N)
        var = sq * (1.0 / N) - mu * mu
        a_ch = gampat[ch] * _rsqrt_scalar(var + EPS)
        b_ch = betpat[ch] - mu * a_ch
        wrow = plsc.load_gather(pbuf, [_splat(3 + ch), colp])
        wpats.append(wrow * a_ch)
        cpat = cpat + wrow * b_ch
    return wpats, cpat


def _xs_phase(c, node0, wpats, cpat, hout_hbm, dinv_hbm, xsout_hbm,
              hbuf, dvbuf, xbuf, sxs, colp, half):
    for u in range(NSUB):
        base = node0 + u * SUB
        pltpu.sync_copy(hout_hbm.at[pl.ds(c * N + base, SUB)], hbuf)
        pltpu.sync_copy(dinv_hbm.at[pl.ds(c * N + base, SUB)], dvbuf)

        @pl.loop(0, VPS)
        def _v(v):
            row = 2 * v + half
            acc = cpat
            for ch in range(C):
                hb = plsc.load_gather(hbuf, [row, _splat(ch)])
                acc = acc + hb * wpats[ch]
            dv = plsc.load_gather(dvbuf, [row, colp])
            plsc.store_scatter(xbuf, [row, colp], acc * dv)

        pltpu.sync_copy(xbuf, sxs.at[pl.ds(base, SUB)])
        pltpu.sync_copy(xbuf, xsout_hbm.at[pl.ds(c * N + base, SUB)])


def _deg_body(dst_hbm, ones_hbm, zeros_hbm, out_hbm, sacc, dbuf, obuf, ssem):
    c = lax.axis_index("c")
    s = lax.axis_index("s")
    row0 = s * RT
    pltpu.sync_copy(zeros_hbm, sacc.at[pl.ds(row0, RT)])
    pltpu.sync_copy(ones_hbm, obuf)
    plsc.subcore_barrier()
    base = (c * NS + s) * TILE_ROWS

    @pl.loop(0, NBULK)
    def _bulk(g):
        rb = base + g * BULK
        pltpu.sync_copy(dst_hbm.at[pl.ds(rb, BULK)], dbuf)

        @pl.loop(0, BULK, step=NBUF)
        def _grp(j0):
            descs = [
                pltpu.async_copy(obuf, sacc.at[dbuf.at[j0 + b]],
                                 ssem.at[b], add=True)
                for b in range(NBUF)
            ]
            for d in descs:
                d.wait()

    plsc.subcore_barrier()
    pltpu.sync_copy(sacc.at[pl.ds(row0, RT)],
                    out_hbm.at[pl.ds(c * N + row0, RT)])


_deg_call = pl.kernel(
    _deg_body,
    out_type=jax.ShapeDtypeStruct((NC * N, C), F32),
    mesh=_MESH,
    compiler_params=_SC_PARAMS,
    scratch_types=[
        pltpu.VMEM_SHARED((N, C), F32),
        pltpu.VMEM((BULK, CHUNK), I32),
        pltpu.VMEM((CHUNK, C), F32),
        pltpu.SemaphoreType.DMA((NBUF,)),
    ],
)


def _k0_body(deg_hbm, xe_hbm, emb_hbm, p_hbm, src_hbm, dst_hbm, z_hbm,
             sout_hbm, hout_hbm, dinvout_hbm, xsout_hbm,
             sxs, sacc, sstat, hbuf, s0buf, s1buf, xbuf, dvbuf, pbuf,
             stb, astb, sbuf, dbuf, gbuf, gsem, ssem, ibuf, embbuf):
    c = lax.axis_index("c")
    s = lax.axis_index("s")
    node0 = s * RT
    pltpu.sync_copy(p_hbm, pbuf)
    pltpu.sync_copy(emb_hbm, embbuf)
    pltpu.sync_copy(z_hbm, sacc.at[pl.ds(node0, RT)])

    iota = lax.iota(I32, 16)
    colp = lax.bitwise_and(iota, _splat(7))
    half = lax.shift_right_logical(iota, _splat(3))

    acc1 = jnp.zeros((16,), F32)
    acc2 = jnp.zeros((16,), F32)
    for u in range(NSUB):
        base = node0 + u * SUB
        pltpu.sync_copy(deg_hbm.at[pl.ds(base, SUB)], s0buf)
        pltpu.sync_copy(deg_hbm.at[pl.ds(N + base, SUB)], s1buf)
        pltpu.sync_copy(xe_hbm.at[pl.ds(base, SUB)], ibuf)

        def _v0(v, carry):
            a1, a2 = carry
            row = 2 * v + half
            xv = plsc.load_gather(ibuf, [row, colp])
            h = plsc.load_gather(embbuf, [xv, colp])
            d0 = plsc.load_gather(s0buf, [row, colp])
            d1 = plsc.load_gather(s1buf, [row, colp])
            dv = _rsqrt(d0 + d1 + 1.0)
            plsc.store_scatter(hbuf, [row, colp], h)
            plsc.store_scatter(dvbuf, [row, colp], dv)
            return (a1 + h, a2 + h * h)

        acc1, acc2 = lax.fori_loop(0, VPS, _v0, (acc1, acc2))
        pltpu.sync_copy(hbuf, hout_hbm.at[pl.ds(c * N + base, SUB)])
        pltpu.sync_copy(dvbuf, dinvout_hbm.at[pl.ds(c * N + base, SUB)])

    wpats, cpat = _stats_and_weights(s, acc1, acc2, colp, pbuf, stb, astb,
                                     sstat)

    _xs_phase(c, node0, wpats, cpat, hout_hbm, dinvout_hbm, xsout_hbm,
              hbuf, dvbuf, xbuf, sxs, colp, half)
    plsc.subcore_barrier()
    _edge_phase(c, s, src_hbm, dst_hbm, sxs, sacc, sbuf, dbuf, gbuf,
                gsem, ssem, sout_hbm)


_k0_call = pl.kernel(
    _k0_body,
    out_type=(
        jax.ShapeDtypeStruct((NC * N, C), F32),
        jax.ShapeDtypeStruct((NC * N, C), F32),
        jax.ShapeDtypeStruct((NC * N, C), F32),
        jax.ShapeDtypeStruct((NC * N, C), F32),
    ),
    mesh=_MESH,
    compiler_params=_SC_PARAMS,
    scratch_types=_EDGE_SCRATCH + [
        pltpu.VMEM((SUB, C), I32),
        pltpu.VMEM((T, C), F32),
    ],
)


def _ki_body(h_hbm, sp_hbm, xsp_hbm, dinv_hbm, p_hbm, src_hbm, dst_hbm,
             z_hbm,
             sout_hbm, hout_hbm, xsout_hbm,
             sxs, sacc, sstat, hbuf, s0buf, s1buf, xbuf, dvbuf, pbuf,
             stb, astb, sbuf, dbuf, gbuf, gsem, ssem):
    c = lax.axis_index("c")
    s = lax.axis_index("s")
    node0 = s * RT
    pltpu.sync_copy(p_hbm, pbuf)
    pltpu.sync_copy(z_hbm, sacc.at[pl.ds(node0, RT)])

    iota = lax.iota(I32, 16)
    colp = lax.bitwise_and(iota, _splat(7))
    half = lax.shift_right_logical(iota, _splat(3))
    bprev = plsc.load_gather(pbuf, [_splat(2), colp])

    acc1 = jnp.zeros((16,), F32)
    acc2 = jnp.zeros((16,), F32)
    for u in range(NSUB):
        base = node0 + u * SUB
        pltpu.sync_copy(h_hbm.at[pl.ds(c * N + base, SUB)], hbuf)
        pltpu.sync_copy(sp_hbm.at[pl.ds(base, SUB)], s0buf)
        pltpu.sync_copy(sp_hbm.at[pl.ds(N + base, SUB)], s1buf)
        pltpu.sync_copy(xsp_hbm.at[pl.ds(c * N + base, SUB)], xbuf)
        pltpu.sync_copy(dinv_hbm.at[pl.ds(c * N + base, SUB)], dvbuf)

        def _vu(v, carry):
            a1, a2 = carry
            row = 2 * v + half
            h = plsc.load_gather(hbuf, [row, colp])
            s0 = plsc.load_gather(s0buf, [row, colp])
            s1 = plsc.load_gather(s1buf, [row, colp])
            xv = plsc.load_gather(xbuf, [row, colp])
            dv = plsc.load_gather(dvbuf, [row, colp])
            hn = jnp.maximum(h + (s0 + s1 + xv) * dv + bprev, 0.0)
            plsc.store_scatter(hbuf, [row, colp], hn)
            return (a1 + hn, a2 + hn * hn)

        acc1, acc2 = lax.fori_loop(0, VPS, _vu, (acc1, acc2))
        pltpu.sync_copy(hbuf, hout_hbm.at[pl.ds(c * N + base, SUB)])

    wpats, cpat = _stats_and_weights(s, acc1, acc2, colp, pbuf, stb, astb,
                                     sstat)
    _xs_phase(c, node0, wpats, cpat, hout_hbm, dinv_hbm, xsout_hbm,
              hbuf, dvbuf, xbuf, sxs, colp, half)
    plsc.subcore_barrier()
    _edge_phase(c, s, src_hbm, dst_hbm, sxs, sacc, sbuf, dbuf, gbuf,
                gsem, ssem, sout_hbm)


_ki_call = pl.kernel(
    _ki_body,
    out_type=(
        jax.ShapeDtypeStruct((NC * N, C), F32),
        jax.ShapeDtypeStruct((NC * N, C), F32),
        jax.ShapeDtypeStruct((NC * N, C), F32),
    ),
    mesh=_MESH,
    compiler_params=_SC_PARAMS,
    scratch_types=_EDGE_SCRATCH,
)


def _kfin_body(h_hbm, sp_hbm, xsp_hbm, dinv_hbm, p_hbm, z_hbm, hout_hbm,
               hbuf, s0buf, s1buf, xbuf, dvbuf, pbuf):
    c = lax.axis_index("c")
    s = lax.axis_index("s")
    node0 = s * RT
    pltpu.sync_copy(p_hbm, pbuf)

    iota = lax.iota(I32, 16)
    colp = lax.bitwise_and(iota, _splat(7))
    half = lax.shift_right_logical(iota, _splat(3))
    bprev = plsc.load_gather(pbuf, [_splat(2), colp])

    @pl.when(jnp.logical_and(c == 1, s == NS - 1))
    def _pad():
        pltpu.sync_copy(z_hbm.at[pl.ds(0, NP - N)],
                        hout_hbm.at[pl.ds(N, NP - N)])

    @pl.when(c == 0)
    def _dense():
        for u in range(NSUB):
            base = node0 + u * SUB
            pltpu.sync_copy(h_hbm.at[pl.ds(base, SUB)], hbuf)
            pltpu.sync_copy(sp_hbm.at[pl.ds(base, SUB)], s0buf)
            pltpu.sync_copy(sp_hbm.at[pl.ds(N + base, SUB)], s1buf)
            pltpu.sync_copy(xsp_hbm.at[pl.ds(base, SUB)], xbuf)
            pltpu.sync_copy(dinv_hbm.at[pl.ds(base, SUB)], dvbuf)

            @pl.loop(0, VPS)
            def _vf(v):
                row = 2 * v + half
                h = plsc.load_gather(hbuf, [row, colp])
                s0 = plsc.load_gather(s0buf, [row, colp])
                s1 = plsc.load_gather(s1buf, [row, colp])
                xv = plsc.load_gather(xbuf, [row, colp])
                dv = plsc.load_gather(dvbuf, [row, colp])
                hn = jnp.maximum(h + (s0 + s1 + xv) * dv + bprev, 0.0)
                plsc.store_scatter(hbuf, [row, colp], hn)

            pltpu.sync_copy(hbuf, hout_hbm.at[pl.ds(base, SUB)])


_kfin_call = pl.kernel(
    _kfin_body,
    out_type=jax.ShapeDtypeStruct((NP, C), F32),
    mesh=_MESH,
    compiler_params=_SC_PARAMS,
    scratch_types=[
        pltpu.VMEM((SUB, C), F32),
        pltpu.VMEM((SUB, C), F32),
        pltpu.VMEM((SUB, C), F32),
        pltpu.VMEM((SUB, C), F32),
        pltpu.VMEM((SUB, C), F32),
        pltpu.VMEM((11, C), F32),
    ],
)


def _pool_body(h_hbm, b_hbm, zeros_hbm, out_hbm, sacc, hbuf, bbuf):
    c = lax.axis_index("c")
    s = lax.axis_index("s")
    gpt = GP // NS
    wid = c * NS + s
    pltpu.sync_copy(zeros_hbm, sacc.at[pl.ds(s * gpt, gpt)])
    pltpu.sync_copy(h_hbm.at[pl.ds(wid * PT, PT)], hbuf)
    pltpu.sync_copy(b_hbm.at[pl.ds(wid * PB, PB)], bbuf)
    plsc.subcore_barrier()

    @pl.loop(0, PB)
    def _step(j):
        pltpu.sync_copy(hbuf.at[pl.ds(j * PCHUNK, PCHUNK)],
                        sacc.at[bbuf.at[j]], add=True)

    plsc.subcore_barrier()
    pltpu.sync_copy(sacc.at[pl.ds(s * gpt, gpt)],
                    out_hbm.at[pl.ds(c * GP + s * gpt, gpt)])


_pool_call = pl.kernel(
    _pool_body,
    out_type=jax.ShapeDtypeStruct((NC * GP, C), F32),
    mesh=_MESH,
    compiler_params=_SC_PARAMS,
    scratch_types=[
        pltpu.VMEM_SHARED((GP, C), F32),
        pltpu.VMEM((PT, C), F32),
        pltpu.VMEM((PB, PCHUNK), I32),
    ],
)


def _mm(a, b):
    return jnp.dot(a, b, preferred_element_type=F32)


def _head_body(p0_ref, p1_ref, hw_ref, hb_ref, ow_ref, ob_ref, out_ref):
    p = p0_ref[...] + p1_ref[...]
    hid = jnp.maximum(_mm(p, hw_ref[...]) + hb_ref[...], 0.0)
    out_ref[...] = _mm(hid, ow_ref[...]) + ob_ref[...]


_head_call = pl.pallas_call(
    _head_body,
    out_shape=jax.ShapeDtypeStruct((GP, 1), F32),
)


def kernel(x, edge_index, batch, emb, bn_gamma, bn_beta, conv_W, conv_b,
           hidden_W, hidden_b, out_W, out_b):
    x = x.astype(I32)
    src2 = edge_index[0].astype(I32).reshape(E // CHUNK, CHUNK)
    dst2 = edge_index[1].astype(I32).reshape(E // CHUNK, CHUNK)

    zrows = jnp.zeros((RT, C), F32)
    ones = jnp.ones((CHUNK, C), F32)
    x_exp = jnp.repeat(x, C).reshape(N, C)

    packs = [
        jnp.concatenate([bn_gamma[i:i + 1], bn_beta[i:i + 1],
                         (conv_b[i - 1:i] if i > 0 else conv_b[0:1]),
                         conv_W[i]], axis=0)
        for i in range(L)
    ]
    fin_pack = jnp.concatenate([bn_gamma[0:1], bn_beta[0:1],
                                conv_b[L - 1:L], conv_W[0]], axis=0)

    deg = _deg_call(dst2, ones, zrows)
    s_parts, h2, dinv2, xs2 = _k0_call(deg, x_exp, emb, packs[0],
                                       src2, dst2, zrows)
    for i in range(1, L):
        s_parts, h2, xs2 = _ki_call(h2, s_parts, xs2, dinv2, packs[i],
                                    src2, dst2, zrows)
    h8p = _kfin_call(h2[:N], s_parts, xs2[:N], dinv2[:N], fin_pack, zrows)

    batchp = jnp.pad(batch.astype(I32), (0, NP - N),
                     constant_values=G).reshape(NP // PCHUNK, PCHUNK)
    pooled = _pool_call(h8p, batchp, jnp.zeros((GP // NS, C), F32))

    out = _head_call(pooled[:GP], pooled[GP:], hidden_W,
                     hidden_b.reshape(1, H), out_W, out_b.reshape(1, 1))
    return out[:G, 0]


# speedup vs baseline: 1.1084x; 1.1084x over previous
import numpy as np
import jax
import jax.numpy as jnp
from jax import lax
from jax.experimental import pallas as pl
from jax.experimental.pallas import tpu as pltpu
from jax.experimental.pallas import tpu_sc as plsc

N = 100000
E = 6400000
G = 1000
C = 8
L = 8
H = 128
T = 6
EPS = 1e-5

NC = 2
NS = 16
NW = NC * NS

CHUNK = 80
RT = N // NS
ET = E // NW
TILE_ROWS = ET // CHUNK
BULK = 125
NBULK = TILE_ROWS // BULK

MR = (N * C) // 128

PCHUNK = 64
NP = ((N + NW * PCHUNK - 1) // (NW * PCHUNK)) * (NW * PCHUNK)
PT = NP // NW
PB = PT // PCHUNK
GP = 1024

_SEL = np.tile(np.eye(C, dtype=np.float32), (16, 1))
_SELT = np.ascontiguousarray(_SEL.T)
_BD = np.kron(np.eye(16, dtype=np.float32),
              np.ones((C, C), dtype=np.float32))

_MESH = plsc.VectorSubcoreMesh(core_axis_name="c", subcore_axis_name="s",
                               num_cores=NC, num_subcores=NS)
_SC_PARAMS = pltpu.CompilerParams(use_tc_tiling_on_sc=False)


NBUF = 5


def _deg_body(dst_hbm, ones_hbm, zeros_hbm, out_hbm, sacc, dbuf, obuf, ssem):
    c = lax.axis_index("c")
    s = lax.axis_index("s")
    row0 = s * RT
    pltpu.sync_copy(zeros_hbm, sacc.at[pl.ds(row0, RT)])
    pltpu.sync_copy(ones_hbm, obuf)
    plsc.subcore_barrier()
    base = (c * NS + s) * TILE_ROWS

    @pl.loop(0, NBULK)
    def _bulk(g):
        rb = base + g * BULK
        pltpu.sync_copy(dst_hbm.at[pl.ds(rb, BULK)], dbuf)

        @pl.loop(0, BULK, step=NBUF)
        def _grp(j0):
            descs = [
                pltpu.async_copy(obuf, sacc.at[dbuf.at[j0 + b]],
                                 ssem.at[b], add=True)
                for b in range(NBUF)
            ]
            for d in descs:
                d.wait()

    plsc.subcore_barrier()
    pltpu.sync_copy(sacc.at[pl.ds(row0, RT)],
                    out_hbm.at[pl.ds(c * N + row0, RT)])


_deg_call = pl.kernel(
    _deg_body,
    out_type=jax.ShapeDtypeStruct((NC * N, C), jnp.float32),
    mesh=_MESH,
    compiler_params=_SC_PARAMS,
    scratch_types=[
        pltpu.VMEM_SHARED((N, C), jnp.float32),
        pltpu.VMEM((BULK, CHUNK), jnp.int32),
        pltpu.VMEM((CHUNK, C), jnp.float32),
        pltpu.SemaphoreType.DMA((NBUF,)),
    ],
)


def _agg_body(xs_hbm, src_hbm, dst_hbm, zeros_hbm, out_hbm,
              sxs, sacc, sbuf, dbuf, gbuf, gsem, ssem):
    c = lax.axis_index("c")
    s = lax.axis_index("s")
    row0 = s * RT
    pltpu.sync_copy(xs_hbm.at[pl.ds(row0, RT)], sxs.at[pl.ds(row0, RT)])
    pltpu.sync_copy(zeros_hbm, sacc.at[pl.ds(row0, RT)])
    plsc.subcore_barrier()
    base = (c * NS + s) * TILE_ROWS

    @pl.loop(0, NBULK)
    def _bulk(g):
        rb = base + g * BULK
        pltpu.sync_copy(src_hbm.at[pl.ds(rb, BULK)], sbuf)
        pltpu.sync_copy(dst_hbm.at[pl.ds(rb, BULK)], dbuf)

        @pl.loop(0, BULK, step=NBUF)
        def _grp(j0):
            gd = [
                pltpu.async_copy(sxs.at[sbuf.at[j0 + b]], gbuf.at[b],
                                 gsem.at[b])
                for b in range(NBUF)
            ]
            sd = []
            for b in range(NBUF):
                gd[b].wait()
                sd.append(
                    pltpu.async_copy(gbuf.at[b], sacc.at[dbuf.at[j0 + b]],
                                     ssem.at[b], add=True))
            for d in sd:
                d.wait()

    plsc.subcore_barrier()
    pltpu.sync_copy(sacc.at[pl.ds(row0, RT)],
                    out_hbm.at[pl.ds(c * N + row0, RT)])


_agg_call = pl.kernel(
    _agg_body,
    out_type=jax.ShapeDtypeStruct((NC * N, C), jnp.float32),
    mesh=_MESH,
    compiler_params=_SC_PARAMS,
    scratch_types=[
        pltpu.VMEM_SHARED((N, C), jnp.float32),
        pltpu.VMEM_SHARED((N, C), jnp.float32),
        pltpu.VMEM((BULK, CHUNK), jnp.int32),
        pltpu.VMEM((BULK, CHUNK), jnp.int32),
        pltpu.VMEM((NBUF, CHUNK, C), jnp.float32),
        pltpu.SemaphoreType.DMA((NBUF,)),
        pltpu.SemaphoreType.DMA((NBUF,)),
    ],
)


def _pool_body(h_hbm, b_hbm, zeros_hbm, out_hbm, sacc, hbuf, bbuf):
    c = lax.axis_index("c")
    s = lax.axis_index("s")
    gpt = GP // NS
    wid = c * NS + s
    pltpu.sync_copy(zeros_hbm, sacc.at[pl.ds(s * gpt, gpt)])
    pltpu.sync_copy(h_hbm.at[pl.ds(wid * PT, PT)], hbuf)
    pltpu.sync_copy(b_hbm.at[pl.ds(wid * PB, PB)], bbuf)
    plsc.subcore_barrier()

    @pl.loop(0, PB)
    def _step(j):
        pltpu.sync_copy(hbuf.at[pl.ds(j * PCHUNK, PCHUNK)],
                        sacc.at[bbuf.at[j]], add=True)

    plsc.subcore_barrier()
    pltpu.sync_copy(sacc.at[pl.ds(s * gpt, gpt)],
                    out_hbm.at[pl.ds(c * GP + s * gpt, gpt)])


_pool_call = pl.kernel(
    _pool_body,
    out_type=jax.ShapeDtypeStruct((NC * GP, C), jnp.float32),
    mesh=_MESH,
    compiler_params=_SC_PARAMS,
    scratch_types=[
        pltpu.VMEM_SHARED((GP, C), jnp.float32),
        pltpu.VMEM((PT, C), jnp.float32),
        pltpu.VMEM((PB, PCHUNK), jnp.int32),
    ],
)


def _mm(a, b):
    return jnp.dot(a, b, preferred_element_type=jnp.float32)


def _make_xs(h, dinv, gamma, beta, W, sel, selt, bd):
    su = _mm(jnp.sum(h, axis=0, keepdims=True), sel)
    sq = _mm(jnp.sum(h * h, axis=0, keepdims=True), sel)
    mu = su / N
    var = sq / N - mu * mu
    a = gamma / jnp.sqrt(var + EPS)
    b = beta - mu * a
    acol = lax.dot_general(sel, a, (((1,), (1,)), ((), ())),
                           preferred_element_type=jnp.float32)
    tw = _mm(_mm(sel, W), selt)
    big_w = acol * tw * bd
    cpat = _mm(_mm(b, W), selt)
    return (_mm(h, big_w) + cpat) * dinv


def _u0_body(xe_ref, embt_ref, deg0_ref, deg1_ref, gamma_ref, beta_ref,
             w_ref, sel_ref, selt_ref, bd_ref, h_ref, dinv_ref, xs_ref):
    xe = xe_ref[...]
    embt = embt_ref[...]
    h = jnp.zeros((MR, 128), jnp.float32)
    for t in range(T):
        h = h + jnp.where(xe == t, embt[t:t + 1, :], 0.0)
    dinv = lax.rsqrt(deg0_ref[...] + deg1_ref[...] + 1.0)
    h_ref[...] = h
    dinv_ref[...] = dinv
    xs_ref[...] = _make_xs(h, dinv, gamma_ref[...], beta_ref[...],
                           w_ref[...], sel_ref[...], selt_ref[...],
                           bd_ref[...])


_u0_call = pl.pallas_call(
    _u0_body,
    out_shape=(
        jax.ShapeDtypeStruct((MR, 128), jnp.float32),
        jax.ShapeDtypeStruct((MR, 128), jnp.float32),
        jax.ShapeDtypeStruct((MR, 128), jnp.float32),
    ),
)


def _ui_body(h_ref, s0_ref, s1_ref, xs_ref, dinv_ref, bprev_ref, gamma_ref,
             beta_ref, w_ref, sel_ref, selt_ref, bd_ref, hn_ref, xsn_ref):
    dinv = dinv_ref[...]
    selt = selt_ref[...]
    bpat = _mm(bprev_ref[...], selt)
    h = jnp.maximum(
        h_ref[...] + (s0_ref[...] + s1_ref[...] + xs_ref[...]) * dinv + bpat,
        0.0)
    hn_ref[...] = h
    xsn_ref[...] = _make_xs(h, dinv, gamma_ref[...], beta_ref[...],
                            w_ref[...], sel_ref[...], selt, bd_ref[...])


_ui_call = pl.pallas_call(
    _ui_body,
    out_shape=(
        jax.ShapeDtypeStruct((MR, 128), jnp.float32),
        jax.ShapeDtypeStruct((MR, 128), jnp.float32),
    ),
)


def _fin_body(h_ref, s0_ref, s1_ref, xs_ref, dinv_ref, bprev_ref, selt_ref,
              hn_ref):
    bpat = _mm(bprev_ref[...], selt_ref[...])
    hn_ref[...] = jnp.maximum(
        h_ref[...]
        + (s0_ref[...] + s1_ref[...] + xs_ref[...]) * dinv_ref[...] + bpat,
        0.0)


_fin_call = pl.pallas_call(
    _fin_body,
    out_shape=jax.ShapeDtypeStruct((MR, 128), jnp.float32),
)


def _head_body(p0_ref, p1_ref, hw_ref, hb_ref, ow_ref, ob_ref, out_ref):
    p = p0_ref[...] + p1_ref[...]
    hid = jnp.maximum(_mm(p, hw_ref[...]) + hb_ref[...], 0.0)
    out_ref[...] = _mm(hid, ow_ref[...]) + ob_ref[...]


_head_call = pl.pallas_call(
    _head_body,
    out_shape=jax.ShapeDtypeStruct((GP, 1), jnp.float32),
)


def kernel(x, edge_index, batch, emb, bn_gamma, bn_beta, conv_W, conv_b,
           hidden_W, hidden_b, out_W, out_b):
    x = x.astype(jnp.int32)
    src2 = edge_index[0].astype(jnp.int32).reshape(E // CHUNK, CHUNK)
    dst2 = edge_index[1].astype(jnp.int32).reshape(E // CHUNK, CHUNK)

    zrows = jnp.zeros((RT, C), jnp.float32)
    ones = jnp.ones((CHUNK, C), jnp.float32)
    sel = jnp.asarray(_SEL)
    selt = jnp.asarray(_SELT)
    bd = jnp.asarray(_BD)

    deg = _deg_call(dst2, ones, zrows)
    deg0 = deg[:N].reshape(MR, 128)
    deg1 = deg[N:].reshape(MR, 128)

    x_exp = jnp.repeat(x, C).reshape(MR, 128)
    embt = jnp.tile(emb, (1, 16))

    h, dinv, xs = _u0_call(x_exp, embt, deg0, deg1, bn_gamma[0:1],
                           bn_beta[0:1], conv_W[0], sel, selt, bd)
    for i in range(1, L):
        s_parts = _agg_call(xs.reshape(N, C), src2, dst2, zrows)
        h, xs = _ui_call(h, s_parts[:N].reshape(MR, 128),
                         s_parts[N:].reshape(MR, 128), xs, dinv,
                         conv_b[i - 1:i], bn_gamma[i:i + 1],
                         bn_beta[i:i + 1], conv_W[i], sel, selt, bd)
    s_parts = _agg_call(xs.reshape(N, C), src2, dst2, zrows)
    h8 = _fin_call(h, s_parts[:N].reshape(MR, 128),
                   s_parts[N:].reshape(MR, 128), xs, dinv,
                   conv_b[L - 1:L], selt)

    h8p = jnp.pad(h8.reshape(N, C), ((0, NP - N), (0, 0)))
    batchp = jnp.pad(batch.astype(jnp.int32), (0, NP - N),
                     constant_values=G).reshape(NP // PCHUNK, PCHUNK)
    pooled = _pool_call(h8p, batchp, jnp.zeros((GP // NS, C), jnp.float32))

    out = _head_call(pooled[:GP], pooled[GP:], hidden_W,
                     hidden_b.reshape(1, H), out_W, out_b.reshape(1, 1))
    return out[:G, 0]
